# baseline (speedup 1.0000x reference)
import functools

import jax
import jax.numpy as jnp
from jax.experimental import pallas as pl
from jax.experimental.pallas import tpu as pltpu
from jax.experimental.pallas import tpu_sc as plsc

D = 384
K = 8192
BR = 256
BC = 2048

_COMMIT = (0.05, 0.25, 0.6)


def _argmin_kernel(nr, nc, scale, z_ref, ct_ref, idx_ref, loss_ref,
                   minacc, idxacc, cnorm_scr, loss_scr):
    i = pl.program_id(0)
    j = pl.program_id(1)
    z = z_ref[...]
    ct = ct_ref[...]

    @pl.when(i == 0)
    def _():
        cnorm_scr[0, pl.ds(j * BC, BC)] = jnp.sum(ct * ct, axis=0)

    dots = jax.lax.dot_general(z, ct, (((1,), (0,)), ((), ())),
                               preferred_element_type=jnp.float32)
    znorm = jnp.sum(z * z, axis=1, keepdims=True)
    cnorm = cnorm_scr[0, pl.ds(j * BC, BC)]
    d = (znorm - 2.0 * dots) + cnorm[None, :]
    cols = jax.lax.broadcasted_iota(jnp.int32, (BR, BC), 1) + j * BC

    @pl.when(j == 0)
    def _():
        minacc[...] = d
        idxacc[...] = cols

    @pl.when(j > 0)
    def _():
        m = minacc[...]
        upd = d < m
        minacc[...] = jnp.where(upd, d, m)
        idxacc[...] = jnp.where(upd, cols, idxacc[...])

    @pl.when(j == nc - 1)
    def _():
        mall = minacc[...]
        mv = jnp.min(mall, axis=1)
        cand = jnp.where(mall == mv[:, None], idxacc[...],
                         jnp.int32(2147483647))
        idx_ref[0, 0, :] = jnp.min(cand, axis=1)
        s = jnp.sum(mv) * scale

        @pl.when(i == 0)
        def _():
            loss_scr[0, 0] = s

        @pl.when(i > 0)
        def _():
            loss_scr[0, 0] = loss_scr[0, 0] + s

        @pl.when(i == nr - 1)
        def _():
            loss_ref[...] = jnp.full((1, 1), loss_scr[0, 0], jnp.float32)


def _vq_argmin(zf, ct, commit):
    n = zf.shape[0]
    nr = n // BR
    nc = K // BC
    scale = commit / (n * D)
    idx3, loss = pl.pallas_call(
        functools.partial(_argmin_kernel, nr, nc, scale),
        grid=(nr, nc),
        in_specs=[
            pl.BlockSpec((BR, D), lambda i, j: (i, 0)),
            pl.BlockSpec((D, BC), lambda i, j: (0, j)),
        ],
        out_specs=[
            pl.BlockSpec((1, 1, BR), lambda i, j: (i, 0, 0)),
            pl.BlockSpec((1, 1), lambda i, j: (0, 0)),
        ],
        out_shape=[
            jax.ShapeDtypeStruct((nr, 1, BR), jnp.int32),
            jax.ShapeDtypeStruct((1, 1), jnp.float32),
        ],
        scratch_shapes=[
            pltpu.VMEM((BR, BC), jnp.float32),
            pltpu.VMEM((BR, BC), jnp.int32),
            pltpu.VMEM((1, K), jnp.float32),
            pltpu.SMEM((1, 1), jnp.float32),
        ],
    )(zf, ct)
    return idx3.reshape(n), loss


def _sc_gather(cb, idx):
    n = idx.shape[0]
    info = plsc.get_sparse_core_info()
    ncores = info.num_cores
    nw = ncores * info.num_subcores
    b_per_w = n // nw
    mesh = plsc.VectorSubcoreMesh(core_axis_name="c", subcore_axis_name="s")

    @functools.partial(
        pl.kernel, mesh=mesh,
        out_type=jax.ShapeDtypeStruct((n, D), cb.dtype),
        scratch_types=[
            pltpu.VMEM((b_per_w,), jnp.int32),
            pltpu.VMEM((b_per_w, D), jnp.float32),
            pltpu.SemaphoreType.DMA,
        ],
    )
    def k(cb_hbm, idx_hbm, out_hbm, idx_v, rows_v, sem):
        wid = jax.lax.axis_index("s") * ncores + jax.lax.axis_index("c")
        base = wid * b_per_w
        pltpu.sync_copy(idx_hbm.at[pl.ds(base, b_per_w)], idx_v)
        pltpu.async_copy(cb_hbm.at[idx_v], rows_v, sem).wait()
        pltpu.sync_copy(rows_v, out_hbm.at[pl.ds(base, b_per_w)])

    return k(cb, idx)


def kernel(l0, l1, l2, cb0, cb1, cb2):
    idxs, losses, qs = [], [], []
    for z, cb, commit in ((l0, cb0, _COMMIT[0]),
                          (l1, cb1, _COMMIT[1]),
                          (l2, cb2, _COMMIT[2])):
        b, t, _ = z.shape
        zf = z.reshape(b * t, D)
        idx, loss = _vq_argmin(zf, cb.T, commit)
        q = _sc_gather(cb, idx)
        idxs.append(idx.reshape(b, t))
        losses.append(loss[0, 0])
        qs.append(q.reshape(b, t, D))
    total_loss = losses[0] + losses[1] + losses[2]
    return (idxs[0], idxs[1], idxs[2], total_loss, qs[0], qs[1], qs[2])


# speedup vs baseline: 1.4103x; 1.4103x over previous
import functools

import jax
import jax.numpy as jnp
from jax.experimental import pallas as pl
from jax.experimental.pallas import tpu as pltpu
from jax.experimental.pallas import tpu_sc as plsc

D = 384
K = 8192
BR = 256
BC = 2048

_COMMIT = (0.05, 0.25, 0.6)


def _argmin_kernel(z_ref, cb_ref, zn_ref, cn_ref, idx_ref):
    z = z_ref[...]
    cb = cb_ref[...]
    dots = jax.lax.dot_general(z, cb, (((1,), (1,)), ((), ())),
                               preferred_element_type=jnp.float32)
    d = (zn_ref[...] - 2.0 * dots) + cn_ref[...]
    mv = jnp.min(d, axis=1)
    cols = jax.lax.broadcasted_iota(jnp.int32, (BR, K), 1)
    cand = jnp.where(d == mv[:, None], cols, jnp.int32(2147483647))
    idx_ref[0, 0, :] = jnp.min(cand, axis=1)


def _vq_argmin(zf, cb, zn, cn):
    n = zf.shape[0]
    nr = n // BR
    idx3 = pl.pallas_call(
        _argmin_kernel,
        grid=(nr,),
        in_specs=[
            pl.BlockSpec((BR, D), lambda i: (i, 0)),
            pl.BlockSpec((K, D), lambda i: (0, 0)),
            pl.BlockSpec((BR, 1), lambda i: (i, 0)),
            pl.BlockSpec((1, K), lambda i: (0, 0)),
        ],
        out_specs=pl.BlockSpec((1, 1, BR), lambda i: (i, 0, 0)),
        out_shape=jax.ShapeDtypeStruct((nr, 1, BR), jnp.int32),
    )(zf, cb, zn, cn)
    return idx3.reshape(n)


def _loss_kernel(nr, scale, z_ref, q_ref, loss_ref, loss_scr):
    i = pl.program_id(0)
    r = z_ref[...] - q_ref[...]
    s = jnp.sum(r * r)

    @pl.when(i == 0)
    def _():
        loss_scr[0, 0] = s

    @pl.when(i > 0)
    def _():
        loss_scr[0, 0] = loss_scr[0, 0] + s

    @pl.when(i == nr - 1)
    def _():
        loss_ref[...] = jnp.full((1, 1), loss_scr[0, 0] * scale,
                                 jnp.float32)


def _loss(zf, q, commit):
    n = zf.shape[0]
    nr = n // BR
    scale = commit / (n * D)
    out = pl.pallas_call(
        functools.partial(_loss_kernel, nr, scale),
        grid=(nr,),
        in_specs=[
            pl.BlockSpec((BR, D), lambda i: (i, 0)),
            pl.BlockSpec((BR, D), lambda i: (i, 0)),
        ],
        out_specs=pl.BlockSpec((1, 1), lambda i: (0, 0)),
        out_shape=jax.ShapeDtypeStruct((1, 1), jnp.float32),
        scratch_shapes=[pltpu.SMEM((1, 1), jnp.float32)],
    )(zf, q)
    return out[0, 0]


def _sc_gather(cb, idx):
    n = idx.shape[0]
    info = plsc.get_sparse_core_info()
    ncores = info.num_cores
    nw = ncores * info.num_subcores
    b_per_w = n // nw
    mesh = plsc.VectorSubcoreMesh(core_axis_name="c", subcore_axis_name="s")

    @functools.partial(
        pl.kernel, mesh=mesh,
        out_type=jax.ShapeDtypeStruct((n, D), cb.dtype),
        scratch_types=[
            pltpu.VMEM((b_per_w,), jnp.int32),
            pltpu.VMEM((b_per_w, D), jnp.float32),
            pltpu.SemaphoreType.DMA,
        ],
    )
    def k(cb_hbm, idx_hbm, out_hbm, idx_v, rows_v, sem):
        wid = jax.lax.axis_index("s") * ncores + jax.lax.axis_index("c")
        base = wid * b_per_w
        pltpu.sync_copy(idx_hbm.at[pl.ds(base, b_per_w)], idx_v)
        pltpu.async_copy(cb_hbm.at[idx_v], rows_v, sem).wait()
        pltpu.sync_copy(rows_v, out_hbm.at[pl.ds(base, b_per_w)])

    return k(cb, idx)


def kernel(l0, l1, l2, cb0, cb1, cb2):
    idxs, losses, qs = [], [], []
    for z, cb, commit in ((l0, cb0, _COMMIT[0]),
                          (l1, cb1, _COMMIT[1]),
                          (l2, cb2, _COMMIT[2])):
        b, t, _ = z.shape
        zf = z.reshape(b * t, D)
        zn = jnp.sum(zf * zf, axis=1, keepdims=True)
        cn = jnp.sum(cb * cb, axis=1)[None, :]
        idx = _vq_argmin(zf, cb, zn, cn)
        q = _sc_gather(cb, idx)
        losses.append(_loss(zf, q, commit))
        idxs.append(idx.reshape(b, t))
        qs.append(q.reshape(b, t, D))
    total_loss = losses[0] + losses[1] + losses[2]
    return (idxs[0], idxs[1], idxs[2], total_loss, qs[0], qs[1], qs[2])
